# Initial kernel scaffold; baseline (speedup 1.0000x reference)
import jax
import jax.numpy as jnp
from jax.experimental import pallas as pl


def kernel(pos_triples, neg_triples, ent_emb, rel_emb, proj_matrix):
    raise NotImplementedError("write your pallas kernel here")


# speedup vs baseline: 1.3324x; 1.3324x over previous
import functools

import jax
import jax.numpy as jnp
from jax import lax
from jax.experimental import pallas as pl
from jax.experimental.pallas import tpu as pltpu
from jax.experimental.pallas import tpu_sc as plsc

_ENT_DIM = 32
_REL_DIM = 32
_B = 16384
_MARGIN = 6.0
_L = 16
_NC = 2
_NS = 16
_NW = _NC * _NS
_BPW = _B // _NW
_G = _BPW // _L


def _rsqrt_nt(x):
    i = plsc.bitcast(x, jnp.int32)
    i = jnp.int32(0x5F3759DF) - lax.shift_right_logical(i, 1)
    y = plsc.bitcast(i, jnp.float32)
    for _ in range(3):
        y = y * (1.5 - ((0.5 * x) * y) * y)
    return y


def _make_sc_kernel():
    mesh = plsc.VectorSubcoreMesh(core_axis_name="c", subcore_axis_name="s")

    @functools.partial(
        pl.kernel,
        out_type=jax.ShapeDtypeStruct((_NW, _L), jnp.float32),
        mesh=mesh,
        compiler_params=pltpu.CompilerParams(needs_layout_passes=False,
                                             use_tc_tiling_on_sc=False),
        scratch_types=[
            pltpu.VMEM((6 * _BPW,), jnp.int32),
            pltpu.VMEM((_L, _ENT_DIM), jnp.float32),
            pltpu.VMEM((_L, _ENT_DIM), jnp.float32),
            pltpu.VMEM((_L, _REL_DIM), jnp.float32),
            pltpu.VMEM((_L, _REL_DIM * _ENT_DIM), jnp.float32),
            pltpu.VMEM((_L, _ENT_DIM), jnp.float32),
            pltpu.VMEM((_L, _ENT_DIM), jnp.float32),
            pltpu.VMEM((_L, _REL_DIM), jnp.float32),
            pltpu.VMEM((_L, _REL_DIM * _ENT_DIM), jnp.float32),
            pltpu.VMEM((_L,), jnp.float32),
            pltpu.SemaphoreType.DMA,
        ],
    )
    def sc_loss(idx_hbm, ent_hbm, rel_hbm, proj_hbm, out_hbm,
                idx_v, eh0, et0, r0, pj0, eh1, et1, r1, pj1, acc_v, sem):
        side_refs = ((eh0, et0, r0, pj0), (eh1, et1, r1, pj1))
        wid = lax.axis_index("s") * _NC + lax.axis_index("c")
        base = wid * _BPW
        lane = lax.iota(jnp.int32, _L)
        cols = [jnp.full((_L,), j, jnp.int32) for j in range(_ENT_DIM)]
        zero = jnp.zeros((_L,), jnp.float32)

        for k in range(6):
            pltpu.sync_copy(idx_hbm.at[k, pl.ds(base, _BPW)],
                            idx_v.at[pl.ds(k * _BPW, _BPW)])

        def distance(refs):
            eh_v, et_v, r_v, proj_v = refs

            def gth(ref, col):
                return plsc.load_gather(ref, [lane, col])

            sh, st = zero, zero
            for j in range(_ENT_DIM):
                hj = gth(eh_v, cols[j])
                tj = gth(et_v, cols[j])
                sh = sh + hj * hj
                st = st + tj * tj
            ih = _rsqrt_nt(sh)
            it = _rsqrt_nt(st)
            xs = tuple(gth(eh_v, cols[j]) * ih - gth(et_v, cols[j]) * it
                       for j in range(_ENT_DIM))

            def inner(i, carry):
                dsq = carry[0]
                xs_ = carry[1:]
                ibase = jnp.full((_L,), i * _ENT_DIM, jnp.int32)
                acc = zero
                for j in range(_ENT_DIM):
                    pj = gth(proj_v, ibase + j)
                    acc = acc + pj * xs_[j]
                ri = gth(r_v, jnp.full((_L,), i, jnp.int32))
                v = acc + ri
                return (dsq + v * v,) + xs_

            dsq = lax.fori_loop(0, _REL_DIM, inner, (zero,) + xs)[0]
            return dsq * _rsqrt_nt(dsq)

        def group(l, acc):
            off = l * _L
            descs = []
            for s in range(2):
                so = off + s * (3 * _BPW)
                hv = idx_v[pl.ds(so, _L)]
                rv = idx_v[pl.ds(so + _BPW, _L)]
                tv = idx_v[pl.ds(so + 2 * _BPW, _L)]
                eh_v, et_v, r_v, proj_v = side_refs[s]
                descs.append(pltpu.async_copy(ent_hbm.at[hv], eh_v, sem))
                descs.append(pltpu.async_copy(ent_hbm.at[tv], et_v, sem))
                descs.append(pltpu.async_copy(rel_hbm.at[rv], r_v, sem))
                descs.append(pltpu.async_copy(proj_hbm.at[rv], proj_v, sem))
            for d in descs:
                d.wait()
            dp = distance(side_refs[0])
            dn = distance(side_refs[1])
            return acc + jnp.maximum(dp - dn + _MARGIN, 0.0)

        acc = lax.fori_loop(0, _G, group, zero)
        acc_v[...] = acc
        pltpu.sync_copy(acc_v, out_hbm.at[wid])

    return sc_loss


_SC_LOSS = _make_sc_kernel()


@jax.jit
def kernel(pos_triples, neg_triples, ent_emb, rel_emb, proj_matrix):
    idx = jnp.concatenate([pos_triples, neg_triples], axis=0).astype(jnp.int32)
    partials = _SC_LOSS(idx, ent_emb, rel_emb, proj_matrix)
    return jnp.sum(partials) / jnp.float32(_B)
